# baseline (speedup 1.0000x reference)
import functools

import jax
import jax.numpy as jnp
from jax import lax
from jax.experimental import pallas as pl
from jax.experimental.pallas import tpu as pltpu
from jax.experimental.pallas import tpu_sc as plsc

N = 1048576
K = 2604
TPAD = 2736
STRIDE = 2737
NC, NS, L = 2, 16, 16
NW = NC * NS
PER_TILE = N // NW
NCHUNK = 8
CHUNK = PER_TILE // NCHUNK
TABS = L * STRIDE
VPAD = 102400
VCHUNK = VPAD // NS

_GATHER_STEPS = (512, 256, 128, 64, 32, 16, 8, 4, 2, 1)

_mesh = plsc.VectorSubcoreMesh(
    core_axis_name="c", subcore_axis_name="s", num_cores=NC, num_subcores=NS
)


@functools.partial(
    pl.kernel,
    out_type=jax.ShapeDtypeStruct((N,), jnp.int32),
    mesh=_mesh,
    scratch_types=[
        pltpu.VMEM((VPAD,), jnp.int32),
        pltpu.VMEM((VCHUNK,), jnp.int32),
        pltpu.VMEM((CHUNK,), jnp.int32),
        pltpu.VMEM((CHUNK,), jnp.int32),
        pltpu.VMEM_SHARED((VPAD,), jnp.int32),
        pltpu.SemaphoreType.DMA,
        pltpu.SemaphoreType.DMA,
        pltpu.SemaphoreType.DMA,
        pltpu.SemaphoreType.DMA,
        pltpu.SemaphoreType.DMA,
    ],
    compiler_params=pltpu.CompilerParams(needs_layout_passes=False),
)
def _sc_searchsorted(y_hbm, tabs_hbm, out_hbm, lut_v, bchunk, ya, yb,
                     lut_sh, tab_sem, ys0, ys1, os0, os1):
    sid = lax.axis_index("s")
    wid = sid * NC + lax.axis_index("c")
    base = wid * PER_TILE
    ybufs = (ya, yb)
    ysems = (ys0, ys1)
    osems = (os0, os1)

    h_tab = pltpu.async_copy(tabs_hbm, lut_v.at[pl.ds(0, TABS)], tab_sem)
    hy = [None] * NCHUNK
    ho = [None] * NCHUNK
    for c in range(2):
        hy[c] = pltpu.async_copy(
            y_hbm.at[pl.ds(base + c * CHUNK, CHUNK)], ybufs[c], ysems[c]
        )
    h_tab.wait()

    lanebase = lax.iota(jnp.int32, L) * STRIDE

    def _splat(i):
        i = min(i, TPAD - 1)
        return plsc.load_gather(lut_v, [jnp.full((L,), i, jnp.int32)])

    tA = _splat(2047)
    tB = [_splat(1023 + 2048 * m) for m in range(2)]

    def _start2(y):
        c1 = tA < y
        pos = jnp.where(c1, 2048, 0).astype(jnp.int32)
        c2 = jnp.where(c1, tB[1], tB[0]) < y
        return jnp.where(c2, pos + 1024, pos)

    def _step(pos, y, p2):
        idx = pos + (p2 - 1)
        if p2 > 128:
            idx = jnp.minimum(idx, TPAD - 1)
        t = plsc.load_gather(lut_v, [idx + lanebase])
        return jnp.where(t < y, pos + p2, pos)

    vbase = sid * VCHUNK
    iot = lax.iota(jnp.int32, L)

    BILV = 4

    @plsc.parallel_loop(0, VCHUNK, L * BILV, unroll=2)
    def _build(i):
        qs = [vbase + i + (L * j) + iot for j in range(BILV)]
        poss = [_start2(q) for q in qs]
        for p2 in _GATHER_STEPS:
            for j in range(BILV):
                poss[j] = _step(poss[j], qs[j], p2)
        for j in range(BILV):
            bchunk[pl.ds(i + L * j, L)] = poss[j]

    pltpu.sync_copy(bchunk, lut_sh.at[pl.ds(vbase, VCHUNK)])
    plsc.subcore_barrier()
    pltpu.sync_copy(lut_sh, lut_v)

    ILV = 4
    for c in range(NCHUNK):
        hy[c].wait()
        y_v = ybufs[c % 2]

        @plsc.parallel_loop(0, CHUNK, L * ILV, unroll=2)
        def _lookup(i, y_v=y_v):
            for j in range(ILV):
                sl = pl.ds(i + L * j, L)
                y_v[sl] = plsc.load_gather(lut_v, [y_v[sl]])

        ho[c] = pltpu.async_copy(
            y_v, out_hbm.at[pl.ds(base + c * CHUNK, CHUNK)], osems[c % 2]
        )
        if c + 2 < NCHUNK:
            ho[c].wait()
            hy[c + 2] = pltpu.async_copy(
                y_hbm.at[pl.ds(base + (c + 2) * CHUNK, CHUNK)],
                ybufs[c % 2],
                ysems[c % 2],
            )
    ho[NCHUNK - 2].wait()
    ho[NCHUNK - 1].wait()


def kernel(y_n, unique_cell_types):
    imax = jnp.iinfo(jnp.int32).max
    tab = jnp.concatenate(
        [
            unique_cell_types.astype(jnp.int32),
            jnp.full((STRIDE - K,), imax, jnp.int32),
        ]
    )
    tabs = jnp.tile(tab, L)
    return _sc_searchsorted(y_n.astype(jnp.int32), tabs)


# speedup vs baseline: 1.0034x; 1.0034x over previous
import functools

import jax
import jax.numpy as jnp
from jax import lax
from jax.experimental import pallas as pl
from jax.experimental.pallas import tpu as pltpu
from jax.experimental.pallas import tpu_sc as plsc

N = 1048576
K = 2604
TPAD = 2736
STRIDE = 2737
NC, NS, L = 2, 16, 16
NW = NC * NS
PER_TILE = N // NW
NCHUNK = 8
CHUNK = PER_TILE // NCHUNK
TABS = L * STRIDE
VPAD = 102400
VCHUNK = VPAD // NS

_GATHER_STEPS = (512, 256, 128, 64, 32, 16, 8, 4, 2, 1)

_mesh = plsc.VectorSubcoreMesh(
    core_axis_name="c", subcore_axis_name="s", num_cores=NC, num_subcores=NS
)


@functools.partial(
    pl.kernel,
    out_type=jax.ShapeDtypeStruct((N,), jnp.int32),
    mesh=_mesh,
    scratch_types=[
        pltpu.VMEM((VPAD,), jnp.int32),
        pltpu.VMEM((VCHUNK,), jnp.int32),
        pltpu.VMEM((CHUNK,), jnp.int32),
        pltpu.VMEM((CHUNK,), jnp.int32),
        pltpu.VMEM_SHARED((VPAD,), jnp.int32),
        pltpu.SemaphoreType.DMA,
        pltpu.SemaphoreType.DMA,
        pltpu.SemaphoreType.DMA,
        pltpu.SemaphoreType.DMA,
        pltpu.SemaphoreType.DMA,
    ],
    compiler_params=pltpu.CompilerParams(needs_layout_passes=False),
)
def _sc_searchsorted(y_hbm, tabs_hbm, out_hbm, lut_v, bchunk, ya, yb,
                     lut_sh, tab_sem, ys0, ys1, os0, os1):
    sid = lax.axis_index("s")
    wid = sid * NC + lax.axis_index("c")
    base = wid * PER_TILE
    ybufs = (ya, yb)
    ysems = (ys0, ys1)
    osems = (os0, os1)

    h_tab = pltpu.async_copy(tabs_hbm, lut_v.at[pl.ds(0, TABS)], tab_sem)
    hy = [None] * NCHUNK
    ho = [None] * NCHUNK
    for c in range(2):
        hy[c] = pltpu.async_copy(
            y_hbm.at[pl.ds(base + c * CHUNK, CHUNK)], ybufs[c], ysems[c]
        )
    h_tab.wait()

    lanebase = lax.iota(jnp.int32, L) * STRIDE

    def _splat(i):
        i = min(i, TPAD - 1)
        return plsc.load_gather(lut_v, [jnp.full((L,), i, jnp.int32)])

    tA = _splat(2047)
    tB = [_splat(1023 + 2048 * m) for m in range(2)]

    def _start2(y):
        c1 = tA < y
        pos = jnp.where(c1, 2048, 0).astype(jnp.int32)
        c2 = jnp.where(c1, tB[1], tB[0]) < y
        return jnp.where(c2, pos + 1024, pos)

    def _step(pos, y, p2):
        idx = pos + (p2 - 1)
        if p2 > 128:
            idx = jnp.minimum(idx, TPAD - 1)
        t = plsc.load_gather(lut_v, [idx + lanebase])
        return jnp.where(t < y, pos + p2, pos)

    vbase = sid * VCHUNK
    iot = lax.iota(jnp.int32, L)

    @plsc.parallel_loop(0, VCHUNK, L, unroll=4)
    def _build(i):
        q = vbase + i + iot
        pos = _start2(q)
        for p2 in _GATHER_STEPS:
            pos = _step(pos, q, p2)
        bchunk[pl.ds(i, L)] = pos

    pltpu.sync_copy(bchunk, lut_sh.at[pl.ds(vbase, VCHUNK)])
    plsc.subcore_barrier()
    pltpu.sync_copy(lut_sh, lut_v)

    ILV = 4
    for c in range(NCHUNK):
        hy[c].wait()
        y_v = ybufs[c % 2]

        @plsc.parallel_loop(0, CHUNK, L * ILV, unroll=2)
        def _lookup(i, y_v=y_v):
            for j in range(ILV):
                sl = pl.ds(i + L * j, L)
                y_v[sl] = plsc.load_gather(lut_v, [y_v[sl]])

        ho[c] = pltpu.async_copy(
            y_v, out_hbm.at[pl.ds(base + c * CHUNK, CHUNK)], osems[c % 2]
        )
        if c + 2 < NCHUNK:
            ho[c].wait()
            hy[c + 2] = pltpu.async_copy(
                y_hbm.at[pl.ds(base + (c + 2) * CHUNK, CHUNK)],
                ybufs[c % 2],
                ysems[c % 2],
            )
    ho[NCHUNK - 2].wait()
    ho[NCHUNK - 1].wait()


def kernel(y_n, unique_cell_types):
    imax = jnp.iinfo(jnp.int32).max
    tab = jnp.concatenate(
        [
            unique_cell_types.astype(jnp.int32),
            jnp.full((STRIDE - K,), imax, jnp.int32),
        ]
    )
    tabs = jnp.tile(tab, L)
    return _sc_searchsorted(y_n.astype(jnp.int32), tabs)
